# baseline (speedup 1.0000x reference)
import functools

import jax
import jax.numpy as jnp
from jax import lax
from jax.experimental import pallas as pl
from jax.experimental.pallas import tpu as pltpu
from jax.experimental.pallas import tpu_sc as plsc

NC = 2
NS = 16
NW = NC * NS
LANES = 16
RCHUNK = 16
GROWS = RCHUNK * 5


def _encode(table, idx, wgt, msk, out, idx_v, w_v, m_v, rows_v,
            g0, g1, o0, o1, *, nchunk, hchunks):
    wid = lax.axis_index("s") * NC + lax.axis_index("c")
    rpw = nchunk * RCHUNK

    pltpu.sync_copy(idx.at[wid], idx_v)
    pltpu.sync_copy(wgt.at[wid], w_v)
    pltpu.sync_copy(msk.at[wid], m_v)

    def issue_gather(c, buf, sem):
        pltpu.async_copy(table.at[idx_v.at[c]], buf, sem)

    def issue_out(c, buf, sem):
        base = pl.multiple_of(wid * (rpw * 5) + c * GROWS, 8)
        pltpu.async_copy(buf, out.at[pl.ds(base, GROWS)], sem)

    def compute(c, buf):
        rb = c * RCHUNK
        wvecs = [w_v[j, pl.ds(rb, LANES)] for j in range(5)]
        mvecs = [m_v[j, pl.ds(rb, LANES)] for j in range(1, 5)]
        for r in range(RCHUNK):
            b5 = r * 5
            ws = [wvecs[j][r] for j in range(5)]

            def h_body(h, _, b5=b5, ws=ws):
                sl = pl.ds(h * LANES, LANES)
                acc = buf[b5, sl] * ws[0]
                for j in range(1, 5):
                    acc = acc + buf[b5 + j, sl] * ws[j]
                buf[b5, sl] = acc
                return 0

            lax.fori_loop(0, hchunks, h_body, 0, unroll=4)

            ms = [mvecs[jj][r] for jj in range(4)]
            anyzero = ms[0] * ms[1] * ms[2] * ms[3]

            @pl.when(anyzero < 0.5)
            def _fixup(b5=b5, ms=ms):
                for jj in range(4):
                    @pl.when(ms[jj] < 0.5)
                    def _zero(b5=b5, jj=jj):
                        def z_body(h, _):
                            buf[b5 + 1 + jj, pl.ds(h * LANES, LANES)] = (
                                jnp.zeros((LANES,), jnp.float32))
                            return 0
                        lax.fori_loop(0, hchunks, z_body, 0, unroll=4)

    issue_gather(0, rows_v.at[0], g0)
    issue_gather(1, rows_v.at[1], g1)

    def chunk_body(c, _):
        p = lax.rem(c, 2)
        buf = rows_v.at[p]

        @pl.when(p == 0)
        def _w0():
            pltpu.make_async_copy(table.at[idx_v.at[c]], rows_v.at[0],
                                  g0).wait()

        @pl.when(p == 1)
        def _w1():
            pltpu.make_async_copy(table.at[idx_v.at[c]], rows_v.at[1],
                                  g1).wait()

        compute(c, buf)

        @pl.when(p == 0)
        def _o0():
            issue_out(c, rows_v.at[0], o0)

            @pl.when(c + 2 < nchunk)
            def _r0():
                pltpu.make_async_copy(rows_v.at[0], out.at[pl.ds(0, GROWS)],
                                      o0).wait()
                issue_gather(c + 2, rows_v.at[0], g0)

        @pl.when(p == 1)
        def _o1():
            issue_out(c, rows_v.at[1], o1)

            @pl.when(c + 2 < nchunk)
            def _r1():
                pltpu.make_async_copy(rows_v.at[1], out.at[pl.ds(0, GROWS)],
                                      o1).wait()
                issue_gather(c + 2, rows_v.at[1], g1)
        return 0

    lax.fori_loop(0, nchunk, chunk_body, 0)

    pltpu.make_async_copy(rows_v.at[0], out.at[pl.ds(0, GROWS)], o0).wait()
    pltpu.make_async_copy(rows_v.at[1], out.at[pl.ds(0, GROWS)], o1).wait()


def kernel(utterance_repre, conversation_repre, session_repre,
           state_transition_matrix, max_conversation_length):
    B, L, H = utterance_repre.shape
    S = session_repre.shape[2]
    BL = B * L
    assert BL % (NW * RCHUNK) == 0 and H % LANES == 0
    rpw = BL // NW
    nchunk = rpw // RCHUNK

    stm = state_transition_matrix.astype(jnp.int32)
    maskf = (stm != 0).astype(jnp.float32)
    pos = jnp.clip(stm - 1, 0, S - 1)
    joff = jnp.array([4, 0, 1, 2, 3], jnp.int32)
    src = (jnp.arange(B, dtype=jnp.int32)[:, None, None] * 5
           + joff[None, None, :]) * S + pos
    idx = src.reshape(NW, nchunk, GROWS)
    order = jnp.cumsum(maskf, axis=2) - 1.0
    wgt = maskf * (order < 4.0).astype(jnp.float32) * 0.25
    wgt = wgt.reshape(NW, rpw, 5).transpose(0, 2, 1)
    msk = maskf.reshape(NW, rpw, 5).transpose(0, 2, 1)

    table = session_repre.reshape(B * 5 * S, H)

    body = functools.partial(_encode, nchunk=nchunk, hchunks=H // LANES)
    out = pl.kernel(
        body,
        out_type=jax.ShapeDtypeStruct((BL * 5, H), jnp.float32),
        mesh=plsc.VectorSubcoreMesh(core_axis_name="c", subcore_axis_name="s",
                                    num_cores=NC, num_subcores=NS),
        scratch_types=[
            pltpu.VMEM((nchunk, GROWS), jnp.int32),
            pltpu.VMEM((5, rpw), jnp.float32),
            pltpu.VMEM((5, rpw), jnp.float32),
            pltpu.VMEM((2, GROWS, H), jnp.float32),
            pltpu.SemaphoreType.DMA,
            pltpu.SemaphoreType.DMA,
            pltpu.SemaphoreType.DMA,
            pltpu.SemaphoreType.DMA,
        ],
    )(table, idx, wgt, msk)
    return out.reshape(B, L, 5, H)
